# baseline (speedup 1.0000x reference)
import functools

import jax
import jax.numpy as jnp
import numpy as np
from jax import lax
from jax.experimental import pallas as pl
from jax.experimental.pallas import tpu as pltpu
from jax.experimental.pallas import tpu_sc as plsc

N = 10000
E = 320000
D = 128
D_EDGE = 16
UNITS = 128
STEPS = 4

NC = 2
NS = 16
NW = NC * NS
LANES = 16

CH = 64
CROWS = CH // 2
PKW = 128
CPT = 80
PKS = 16
NCH = 2 * CPT
EPT = NCH * CH
E_PAD = NW * EPT
IDX_ROWS = NW * CPT
BE = 4096

ACC_ROWS = 10112
DUMMY_ROW = N + 8
ZROWS = 632

VPR = D // LANES


def _pack_bf16(r):
    t = lax.bitcast_convert_type(r, jnp.int32)
    b = (t + 0x7FFF + ((t >> 16) & 1)) >> 16
    parts = []
    for g in range(4):
        lo = b[:, 32 * g:32 * g + 16] & 0xFFFF
        hi = b[:, 32 * g + 16:32 * g + 32] << 16
        parts.append(lo | hi)
    return jnp.concatenate(parts, axis=1)


def _edge_perm():
    n = np.arange(E_PAD)
    k = n % CH
    rr = k % CROWS
    is_hi = (k >= CROWS).astype(np.int64)
    row = CROWS * (n // CH) + rr
    orig = (BE * (row // (BE // 2)) + row % (BE // 2)
            + (BE // 2) * is_hi)
    return orig.astype(np.int32)


_EDGE_PERM = _edge_perm()


def _x0_body(nf_ref, w_ref, b_ref, out_ref):
    out_ref[...] = (
        jnp.dot(nf_ref[...], w_ref[...], preferred_element_type=jnp.float32)
        + b_ref[...])


def _x0_call(nf, w0, b0):
    blk = 2000
    return pl.pallas_call(
        _x0_body,
        grid=(N // blk,),
        in_specs=[
            pl.BlockSpec((blk, D), lambda i: (i, 0)),
            pl.BlockSpec((D, UNITS), lambda i: (0, 0)),
            pl.BlockSpec((1, UNITS), lambda i: (0, 0)),
        ],
        out_specs=pl.BlockSpec((blk, UNITS), lambda i: (i, 0)),
        out_shape=jax.ShapeDtypeStruct((N, UNITS), jnp.float32),
    )(nf, w0, b0.reshape(1, UNITS))


def _ep_body(ef_ref, we_ref, be_ref, o0, o1, o2, o3):
    ef = ef_ref[...]
    outs = (o0, o1, o2, o3)
    for s in range(STEPS):
        r = (jnp.dot(ef, we_ref[s], preferred_element_type=jnp.float32)
             + be_ref[s][None, :])
        outs[s][...] = jnp.concatenate(
            [_pack_bf16(r[:BE // 2]), _pack_bf16(r[BE // 2:])], axis=1)


def _ep_call(ef_pad, we, be):
    sds = jax.ShapeDtypeStruct((E_PAD // 2, UNITS), jnp.int32)
    return pl.pallas_call(
        _ep_body,
        grid=(E_PAD // BE,),
        in_specs=[
            pl.BlockSpec((BE, D_EDGE), lambda i: (i, 0)),
            pl.BlockSpec((STEPS, D_EDGE, UNITS), lambda i: (0, 0, 0)),
            pl.BlockSpec((STEPS, UNITS), lambda i: (0, 0)),
        ],
        out_specs=[pl.BlockSpec((BE // 2, UNITS), lambda i: (i, 0))] * STEPS,
        out_shape=[sds] * STEPS,
    )(ef_pad, we, be)


def _upd_body(x_ref, agg_ref, w_ref, b_ref, eps_ref, out_ref):
    h = (1.0 + eps_ref[0, 0]) * x_ref[...] + agg_ref[0] + agg_ref[1]
    out_ref[...] = (
        jnp.dot(h, w_ref[...], preferred_element_type=jnp.float32) + b_ref[...])


def _upd_call(x, agg, wn, bn, eps_s):
    blk = 2000
    return pl.pallas_call(
        _upd_body,
        grid=(N // blk,),
        in_specs=[
            pl.BlockSpec((blk, UNITS), lambda i: (i, 0)),
            pl.BlockSpec((NC, blk, UNITS), lambda i: (0, i, 0)),
            pl.BlockSpec((UNITS, UNITS), lambda i: (0, 0)),
            pl.BlockSpec((1, UNITS), lambda i: (0, 0)),
            pl.BlockSpec(memory_space=pltpu.SMEM),
        ],
        out_specs=pl.BlockSpec((blk, UNITS), lambda i: (i, 0)),
        out_shape=jax.ShapeDtypeStruct((N, UNITS), jnp.float32),
    )(x, agg, wn, bn.reshape(1, UNITS), eps_s)


def _sc_edge_body(x_hbm, ep_hbm, pk_hbm, out_hbm,
                  pk, srcu, dstu, epA, gxA, epB, gxB, msgA, msgB, acc,
                  semEA, semEB, semGA, semGB, semSA, semSB):
    c = lax.axis_index("c")
    s = lax.axis_index("s")
    wid = c * NS + s
    base_row = wid * CPT
    crow_base = wid * (EPT // 2)

    eps_b = (epA, epB)
    gxs_b = (gxA, gxB)
    msgs_b = (msgA, msgB)
    semE = (semEA, semEB)
    semG = (semGA, semGB)
    semS = (semSA, semSB)
    himask = jnp.int32(-65536)

    zv = jnp.zeros((LANES,), jnp.float32)

    def _zrow(r, carry):
        for k in range(VPR):
            msgA[r, pl.ds(k * LANES, LANES)] = zv
        return carry

    lax.fori_loop(0, CH, _zrow, 0)
    for t in range(ZROWS // CH):
        pltpu.sync_copy(msgA, acc.at[pl.ds(s * ZROWS + t * CH, CH)])
    pltpu.sync_copy(msgA.at[pl.ds(0, ZROWS % CH)],
                    acc.at[pl.ds(s * ZROWS + (ZROWS // CH) * CH, ZROWS % CH)])
    plsc.subcore_barrier()


    def stage_pk(piece):
        pltpu.sync_copy(pk_hbm.at[pl.ds(base_row + piece * PKS, PKS)], pk)

    def unpack(j):
        r = (j // 2) % PKS
        h = (j % 2) * CH
        q = j % 4
        for k in range(CH // LANES):
            v = pk[r, pl.ds(h + k * LANES, LANES)]
            srcu[q, pl.ds(k * LANES, LANES)] = v & 0xFFFF
            dstu[q, pl.ds(k * LANES, LANES)] = v >> 16

    def start_eg(j, par):
        q = j % 4
        pltpu.make_async_copy(
            ep_hbm.at[pl.ds(crow_base + j * CROWS, CROWS)], eps_b[par],
            semE[par]).start()
        pltpu.make_async_copy(
            x_hbm.at[srcu.at[q]], gxs_b[par], semG[par]).start()

    def wait_eg(j, par):
        q = j % 4
        pltpu.make_async_copy(
            ep_hbm.at[pl.ds(crow_base + j * CROWS, CROWS)], eps_b[par],
            semE[par]).wait()
        pltpu.make_async_copy(
            x_hbm.at[srcu.at[q]], gxs_b[par], semG[par]).wait()

    def compute(par):
        ep_v, gx_v, ms_v = eps_b[par], gxs_b[par], msgs_b[par]

        def _row(rr, inner):
            for half in range(2):
                mrow = rr + half * CROWS
                for g in range(4):
                    ei = ep_v[rr, pl.ds(half * 64 + g * 16, 16)]
                    lo = (plsc.bitcast(ei << 16, jnp.float32)
                          + gx_v[mrow, pl.ds(g * 32, LANES)])
                    hi = (plsc.bitcast(ei & himask, jnp.float32)
                          + gx_v[mrow, pl.ds(g * 32 + LANES, LANES)])
                    ms_v[mrow, pl.ds(g * 32, LANES)] = jnp.maximum(lo, 0.0)
                    ms_v[mrow, pl.ds(g * 32 + LANES, LANES)] = (
                        jnp.maximum(hi, 0.0))
            return inner

        lax.fori_loop(0, CROWS, _row, 0)

    def start_sc(j, par):
        q = j % 4
        pltpu.make_async_copy(
            msgs_b[par], acc.at[dstu.at[q]], semS[par]).start(add=True)

    def wait_sc(j, par):
        q = j % 4
        pltpu.make_async_copy(
            msgs_b[par], acc.at[dstu.at[q]], semS[par]).wait()

    stage_pk(0)
    unpack(0)
    start_eg(0, 0)
    unpack(1)
    start_eg(1, 1)

    def _pair(p, carry):
        j0 = 2 * p
        j1 = j0 + 1

        @pl.when(j0 >= 2)
        def _():
            wait_sc(j0 - 2, 0)
        wait_eg(j0, 0)
        compute(0)
        start_sc(j0, 0)

        @pl.when((j0 + 2 < NCH) & ((j0 + 2) % (2 * PKS) == 0))
        def _():
            stage_pk((j0 + 2) // (2 * PKS))

        @pl.when(j0 + 2 < NCH)
        def _():
            unpack(j0 + 2)
            start_eg(j0 + 2, 0)

        @pl.when(j1 >= 2)
        def _():
            wait_sc(j1 - 2, 1)
        wait_eg(j1, 1)
        compute(1)
        start_sc(j1, 1)

        @pl.when(j1 + 2 < NCH)
        def _():
            unpack(j1 + 2)
            start_eg(j1 + 2, 1)

        return carry

    lax.fori_loop(0, NCH // 2, _pair, 0)
    wait_sc(NCH - 2, 0)
    wait_sc(NCH - 1, 1)

    plsc.subcore_barrier()
    pltpu.sync_copy(acc.at[pl.ds(s * ZROWS, ZROWS)],
                    out_hbm.at[c].at[pl.ds(s * ZROWS, ZROWS)])


def _sc_edge_call(x, ep, pk2d):
    mesh = plsc.VectorSubcoreMesh(core_axis_name="c", subcore_axis_name="s")
    kern = functools.partial(
        pl.kernel,
        mesh=mesh,
        compiler_params=pltpu.CompilerParams(needs_layout_passes=False),
        out_type=jax.ShapeDtypeStruct((NC, ACC_ROWS, UNITS), jnp.float32),
        scratch_types=[
            pltpu.VMEM((PKS, PKW), jnp.int32),
            pltpu.VMEM((4, CH), jnp.int32),
            pltpu.VMEM((4, CH), jnp.int32),
            pltpu.VMEM((CROWS, UNITS), jnp.int32),
            pltpu.VMEM((CH, UNITS), jnp.float32),
            pltpu.VMEM((CROWS, UNITS), jnp.int32),
            pltpu.VMEM((CH, UNITS), jnp.float32),
            pltpu.VMEM((CH, UNITS), jnp.float32),
            pltpu.VMEM((CH, UNITS), jnp.float32),
            pltpu.VMEM_SHARED((ACC_ROWS, UNITS), jnp.float32),
            pltpu.SemaphoreType.DMA,
            pltpu.SemaphoreType.DMA,
            pltpu.SemaphoreType.DMA,
            pltpu.SemaphoreType.DMA,
            pltpu.SemaphoreType.DMA,
            pltpu.SemaphoreType.DMA,
        ],
    )(_sc_edge_body)
    return kern(x, ep, pk2d)


def kernel(node_feature, edge_feature, edge_src, edge_dst,
           W0, b0, We, be, Wn, bn, eps):
    ef_pad = jnp.pad(edge_feature, ((0, E_PAD - E), (0, 0)))
    pk = edge_src | (edge_dst << 16)
    pk_pad = jnp.concatenate(
        [pk, jnp.full((E_PAD - E,), DUMMY_ROW << 16, jnp.int32)])
    pk2d = jnp.take(pk_pad, _EDGE_PERM).reshape(IDX_ROWS, PKW)

    x = _x0_call(node_feature, W0, b0)
    ep_list = _ep_call(ef_pad, We, be)

    feats = [x]
    for i in range(STEPS):
        agg = _sc_edge_call(x, ep_list[i], pk2d)
        x = _upd_call(x, agg, Wn[i], bn[i], eps[i].reshape(1, 1))
        feats.append(x)
    return jnp.stack(feats, axis=-2)


# speedup vs baseline: 1.2220x; 1.2220x over previous
import functools

import jax
import jax.numpy as jnp
from jax import lax
from jax.experimental import pallas as pl
from jax.experimental.pallas import tpu as pltpu
from jax.experimental.pallas import tpu_sc as plsc

N = 10000
E = 320000
D = 128
D_EDGE = 16
UNITS = 128
STEPS = 4

NC = 2
NS = 16
NW = NC * NS
LANES = 16

CH = 64
PKW = 128
CPT = 80
NCH = 2 * CPT
E_PAD = NW * CPT * PKW
IDX_ROWS = NW * CPT

ACC_ROWS = 10240
DUMMY_ROW = N + 8
ZROWS = 640

VPR = D // LANES


def _x0_body(nf_ref, w_ref, b_ref, out_ref):
    out_ref[...] = (
        jnp.dot(nf_ref[...], w_ref[...], preferred_element_type=jnp.float32)
        + b_ref[...]
    )


def _x0_call(nf, w0, b0):
    blk = 2000
    return pl.pallas_call(
        _x0_body,
        grid=(N // blk,),
        in_specs=[
            pl.BlockSpec((blk, D), lambda i: (i, 0)),
            pl.BlockSpec((D, UNITS), lambda i: (0, 0)),
            pl.BlockSpec((1, UNITS), lambda i: (0, 0)),
        ],
        out_specs=pl.BlockSpec((blk, UNITS), lambda i: (i, 0)),
        out_shape=jax.ShapeDtypeStruct((N, UNITS), jnp.float32),
    )(nf, w0, b0.reshape(1, UNITS))


def _ep_body(ef_ref, we_ref, be_ref, o0, o1, o2, o3):
    ef = ef_ref[...]
    outs = (o0, o1, o2, o3)
    for s in range(STEPS):
        outs[s][...] = (
            jnp.dot(ef, we_ref[s], preferred_element_type=jnp.float32)
            + be_ref[s][None, :]
        )


def _ep_call(ef_pad, we, be):
    blk = 4096
    sds = jax.ShapeDtypeStruct((E_PAD, UNITS), jnp.float32)
    return pl.pallas_call(
        _ep_body,
        grid=(E_PAD // blk,),
        in_specs=[
            pl.BlockSpec((blk, D_EDGE), lambda i: (i, 0)),
            pl.BlockSpec((STEPS, D_EDGE, UNITS), lambda i: (0, 0, 0)),
            pl.BlockSpec((STEPS, UNITS), lambda i: (0, 0)),
        ],
        out_specs=[pl.BlockSpec((blk, UNITS), lambda i: (i, 0))] * STEPS,
        out_shape=[sds] * STEPS,
    )(ef_pad, we, be)


def _upd_body(x_ref, agg_ref, w_ref, b_ref, eps_ref, out_ref):
    h = (1.0 + eps_ref[0, 0]) * x_ref[...] + agg_ref[0] + agg_ref[1]
    out_ref[...] = (
        jnp.dot(h, w_ref[...], preferred_element_type=jnp.float32) + b_ref[...]
    )


def _upd_call(x, agg, wn, bn, eps_s):
    blk = 2000
    return pl.pallas_call(
        _upd_body,
        grid=(N // blk,),
        in_specs=[
            pl.BlockSpec((blk, UNITS), lambda i: (i, 0)),
            pl.BlockSpec((NC, blk, UNITS), lambda i: (0, i, 0)),
            pl.BlockSpec((UNITS, UNITS), lambda i: (0, 0)),
            pl.BlockSpec((1, UNITS), lambda i: (0, 0)),
            pl.BlockSpec(memory_space=pltpu.SMEM),
        ],
        out_specs=pl.BlockSpec((blk, UNITS), lambda i: (i, 0)),
        out_shape=jax.ShapeDtypeStruct((N, UNITS), jnp.float32),
    )(x, agg, wn, bn.reshape(1, UNITS), eps_s)


def _sc_edge_body(x_hbm, ep_hbm, pk_hbm, out_hbm,
                  pk, srcu, dstu, epA, gxA, epB, gxB, acc,
                  semEA, semEB, semGA, semGB, semSA, semSB):
    c = lax.axis_index("c")
    s = lax.axis_index("s")
    wid = c * NS + s
    base_row = wid * CPT
    ebase = base_row * PKW

    eps_b = (epA, epB)
    gxs_b = (gxA, gxB)
    semE = (semEA, semEB)
    semG = (semGA, semGB)
    semS = (semSA, semSB)

    zv = jnp.zeros((LANES,), jnp.float32)

    def _zrow(r, carry):
        for k in range(VPR):
            epA[r, pl.ds(k * LANES, LANES)] = zv
        return carry

    lax.fori_loop(0, CH, _zrow, 0)
    for t in range(ZROWS // CH):
        pltpu.sync_copy(epA, acc.at[pl.ds(s * ZROWS + t * CH, CH)])
    plsc.subcore_barrier()

    pltpu.sync_copy(pk_hbm.at[pl.ds(base_row, CPT)], pk)


    def unpack(j):
        r = j // 2
        h = (j % 2) * CH
        q = j % 4
        for k in range(CH // LANES):
            v = pk[r, pl.ds(h + k * LANES, LANES)]
            srcu[q, pl.ds(k * LANES, LANES)] = v & 0xFFFF
            dstu[q, pl.ds(k * LANES, LANES)] = v >> 16

    def start_eg(j, par):
        q = j % 4
        pltpu.make_async_copy(
            ep_hbm.at[pl.ds(ebase + j * CH, CH)], eps_b[par], semE[par]
        ).start()
        pltpu.make_async_copy(
            x_hbm.at[srcu.at[q]], gxs_b[par], semG[par]).start()

    def wait_eg(j, par):
        q = j % 4
        pltpu.make_async_copy(
            ep_hbm.at[pl.ds(ebase + j * CH, CH)], eps_b[par], semE[par]
        ).wait()
        pltpu.make_async_copy(
            x_hbm.at[srcu.at[q]], gxs_b[par], semG[par]).wait()

    def compute(par):
        ep_v, gx_v = eps_b[par], gxs_b[par]

        def _row(r, inner):
            for k in range(VPR):
                sl = pl.ds(k * LANES, LANES)
                ep_v[r, sl] = jnp.maximum(ep_v[r, sl] + gx_v[r, sl], 0.0)
            return inner

        lax.fori_loop(0, CH, _row, 0)

    def start_sc(j, par):
        q = j % 4
        pltpu.make_async_copy(
            eps_b[par], acc.at[dstu.at[q]], semS[par]).start(add=True)

    def wait_sc(j, par):
        q = j % 4
        pltpu.make_async_copy(
            eps_b[par], acc.at[dstu.at[q]], semS[par]).wait()

    unpack(0)
    start_eg(0, 0)
    unpack(1)
    start_eg(1, 1)

    def _pair(p, carry):
        j0 = 2 * p
        j1 = j0 + 1

        @pl.when(j0 >= 2)
        def _():
            wait_sc(j0 - 2, 0)
        wait_eg(j0, 0)
        compute(0)
        start_sc(j0, 0)

        @pl.when(j0 + 2 < NCH)
        def _():
            unpack(j0 + 2)
            start_eg(j0 + 2, 0)

        @pl.when(j1 >= 2)
        def _():
            wait_sc(j1 - 2, 1)
        wait_eg(j1, 1)
        compute(1)
        start_sc(j1, 1)

        @pl.when(j1 + 2 < NCH)
        def _():
            unpack(j1 + 2)
            start_eg(j1 + 2, 1)

        return carry

    lax.fori_loop(0, NCH // 2, _pair, 0)
    wait_sc(NCH - 2, 0)
    wait_sc(NCH - 1, 1)

    plsc.subcore_barrier()
    pltpu.sync_copy(acc.at[pl.ds(s * ZROWS, ZROWS)],
                    out_hbm.at[c].at[pl.ds(s * ZROWS, ZROWS)])


def _sc_edge_call(x, ep, pk2d):
    mesh = plsc.VectorSubcoreMesh(core_axis_name="c", subcore_axis_name="s")
    kern = functools.partial(
        pl.kernel,
        mesh=mesh,
        out_type=jax.ShapeDtypeStruct((NC, ACC_ROWS, UNITS), jnp.float32),
        scratch_types=[
            pltpu.VMEM((CPT, PKW), jnp.int32),
            pltpu.VMEM((4, CH), jnp.int32),
            pltpu.VMEM((4, CH), jnp.int32),
            pltpu.VMEM((CH, UNITS), jnp.float32),
            pltpu.VMEM((CH, UNITS), jnp.float32),
            pltpu.VMEM((CH, UNITS), jnp.float32),
            pltpu.VMEM((CH, UNITS), jnp.float32),
            pltpu.VMEM_SHARED((ACC_ROWS, UNITS), jnp.float32),
            pltpu.SemaphoreType.DMA,
            pltpu.SemaphoreType.DMA,
            pltpu.SemaphoreType.DMA,
            pltpu.SemaphoreType.DMA,
            pltpu.SemaphoreType.DMA,
            pltpu.SemaphoreType.DMA,
        ],
    )(_sc_edge_body)
    return kern(x, ep, pk2d)


def kernel(node_feature, edge_feature, edge_src, edge_dst,
           W0, b0, We, be, Wn, bn, eps):
    ef_pad = jnp.pad(edge_feature, ((0, E_PAD - E), (0, 0)))
    pk = edge_src | (edge_dst << 16)
    pk2d = jnp.concatenate(
        [pk, jnp.full((E_PAD - E,), DUMMY_ROW << 16, jnp.int32)]
    ).reshape(IDX_ROWS, PKW)

    x = _x0_call(node_feature, W0, b0)
    ep_list = _ep_call(ef_pad, We, be)

    feats = [x]
    for i in range(STEPS):
        agg = _sc_edge_call(x, ep_list[i], pk2d)
        x = _upd_call(x, agg, Wn[i], bn[i], eps[i].reshape(1, 1))
        feats.append(x)
    return jnp.stack(feats, axis=-2)
